# baseline (speedup 1.0000x reference)
import functools

import jax
import jax.numpy as jnp
from jax import lax
from jax.experimental import pallas as pl
from jax.experimental.pallas import tpu as pltpu
from jax.experimental.pallas import tpu_sc as plsc


def _proj_body(t_ref, wt_ref, b_ref, o_ref):
    o_ref[...] = (
        jnp.dot(t_ref[...], wt_ref[...], preferred_element_type=jnp.float32)
        + b_ref[...]
    )


def _project_table(table, W, b):
    V, D = table.shape
    t2 = table.reshape(V // 2, 2 * D)
    Wt = W.T
    W2 = jnp.zeros((2 * D, 2 * D), jnp.float32)
    W2 = W2.at[:D, :D].set(Wt).at[D:, D:].set(Wt)
    b2 = jnp.concatenate([b, b]).reshape(1, 2 * D)
    BLK = 2000
    assert (V // 2) % BLK == 0
    proj2 = pl.pallas_call(
        _proj_body,
        grid=(V // 2 // BLK,),
        in_specs=[
            pl.BlockSpec((BLK, 2 * D), lambda i: (i, 0)),
            pl.BlockSpec((2 * D, 2 * D), lambda i: (0, 0)),
            pl.BlockSpec((1, 2 * D), lambda i: (0, 0)),
        ],
        out_specs=pl.BlockSpec((BLK, 2 * D), lambda i: (i, 0)),
        out_shape=jax.ShapeDtypeStruct((V // 2, 2 * D), jnp.float32),
    )(t2, W2, b2)
    return proj2.reshape(V, D)


_BLK = 128


def _make_gather(V, D, NW, NC, n_blk):
    mesh = plsc.VectorSubcoreMesh(core_axis_name="c", subcore_axis_name="s")

    @functools.partial(
        pl.kernel,
        mesh=mesh,
        out_type=jax.ShapeDtypeStruct((NW * n_blk * _BLK, D), jnp.float32),
        scratch_types=[
            pltpu.VMEM((n_blk, _BLK), jnp.int32),
            pltpu.VMEM((2, _BLK, D), jnp.float32),
            pltpu.SemaphoreType.DMA,
            pltpu.SemaphoreType.DMA,
            pltpu.SemaphoreType.DMA,
            pltpu.SemaphoreType.DMA,
        ],
        compiler_params=pltpu.CompilerParams(use_tc_tiling_on_sc=False),
    )
    def gather(tab_hbm, idx_hbm, out_hbm, idx_v, rows_v, g0, g1, o0, o1):
        gsem = (g0, g1)
        osem = (o0, o1)
        wid = lax.axis_index("s") * NC + lax.axis_index("c")
        pltpu.sync_copy(idx_hbm.at[wid], idx_v)
        base_pair = wid * (n_blk * _BLK)
        pltpu.async_copy(tab_hbm.at[idx_v.at[0]], rows_v.at[0], g0)

        def step(j, s):
            pltpu.make_async_copy(
                tab_hbm.at[idx_v.at[j]], rows_v.at[s], gsem[s]
            ).wait()

            @pl.when(j >= 1)
            def _():
                pltpu.make_async_copy(
                    rows_v.at[1 - s],
                    out_hbm.at[pl.ds(base_pair + (j - 1) * _BLK, _BLK)],
                    osem[1 - s],
                ).wait()

            @pl.when(j < n_blk - 1)
            def _():
                pltpu.async_copy(
                    tab_hbm.at[idx_v.at[j + 1]], rows_v.at[1 - s], gsem[1 - s]
                )

            pltpu.async_copy(
                rows_v.at[s],
                out_hbm.at[pl.ds(base_pair + j * _BLK, _BLK)],
                osem[s],
            )

        def loop(i, carry):
            step(2 * i, 0)
            step(2 * i + 1, 1)
            return carry

        lax.fori_loop(0, n_blk // 2, loop, 0)
        pltpu.make_async_copy(
            rows_v.at[1],
            out_hbm.at[pl.ds(base_pair + (n_blk - 1) * _BLK, _BLK)],
            o1,
        ).wait()

    return gather


_BB = 256


def _trans_body(g_ref, eye_ref, o_ref):
    nlp = g_ref.shape[0] // _BB
    g3 = g_ref.reshape(_BB, nlp, g_ref.shape[1])
    eye = eye_ref[...]
    for lp in range(nlp):
        y = g3[:, lp, :]
        yt = lax.dot_general(
            y, eye, (((0,), (0,)), ((), ())),
            preferred_element_type=jnp.float32,
            precision=lax.Precision.HIGHEST,
        )
        o_ref[2 * lp, :, :] = yt[:64, :]
        o_ref[2 * lp + 1, :, :] = yt[64:, :]


def _to_entry_layout(g2, B, L, D):
    eye = jnp.eye(_BB, dtype=jnp.float32)
    out3 = pl.pallas_call(
        _trans_body,
        grid=(B // _BB,),
        in_specs=[
            pl.BlockSpec((_BB * L // 2, 2 * D), lambda i: (i, 0)),
            pl.BlockSpec((_BB, _BB), lambda i: (0, 0)),
        ],
        out_specs=pl.BlockSpec((L, D, _BB), lambda i: (0, 0, i)),
        out_shape=jax.ShapeDtypeStruct((L, D, B), jnp.float32),
    )(g2, eye)
    return out3.transpose(2, 0, 1)


def kernel(color_indices, table, W, b):
    B, L = color_indices.shape
    V, D = table.shape
    info = plsc.get_sparse_core_info()
    NC, NS = info.num_cores, info.num_subcores
    NW = NC * NS
    total = B * L
    assert total % (NW * _BLK) == 0
    n_blk = total // (NW * _BLK)

    proj = _project_table(table, W, b)
    idx = color_indices.astype(jnp.int32).reshape(NW, n_blk, _BLK)
    g = _make_gather(V, D, NW, NC, n_blk)(proj, idx)
    g2 = g.reshape(B * L // 2, 2 * D)
    return _to_entry_layout(g2, B, L, D)


# speedup vs baseline: 1.1024x; 1.1024x over previous
import functools

import jax
import jax.numpy as jnp
from jax import lax
from jax.experimental import pallas as pl
from jax.experimental.pallas import tpu as pltpu
from jax.experimental.pallas import tpu_sc as plsc


def _proj_body(t_ref, wt_ref, b_ref, o_ref):
    o_ref[...] = (
        jnp.dot(t_ref[...], wt_ref[...], preferred_element_type=jnp.float32)
        + b_ref[...]
    )


def _project_table(table, W, b):
    V, D = table.shape
    t2 = table.reshape(V // 2, 2 * D)
    Wt = W.T
    W2 = jnp.zeros((2 * D, 2 * D), jnp.float32)
    W2 = W2.at[:D, :D].set(Wt).at[D:, D:].set(Wt)
    b2 = jnp.concatenate([b, b]).reshape(1, 2 * D)
    BLK = 2000
    assert (V // 2) % BLK == 0
    proj2 = pl.pallas_call(
        _proj_body,
        grid=(V // 2 // BLK,),
        in_specs=[
            pl.BlockSpec((BLK, 2 * D), lambda i: (i, 0)),
            pl.BlockSpec((2 * D, 2 * D), lambda i: (0, 0)),
            pl.BlockSpec((1, 2 * D), lambda i: (0, 0)),
        ],
        out_specs=pl.BlockSpec((BLK, 2 * D), lambda i: (i, 0)),
        out_shape=jax.ShapeDtypeStruct((V // 2, 2 * D), jnp.float32),
    )(t2, W2, b2)
    return proj2.reshape(V, D)


_BLK = 128


def _make_gather(V, D, NW, NC, n_blk):
    mesh = plsc.VectorSubcoreMesh(core_axis_name="c", subcore_axis_name="s")

    @functools.partial(
        pl.kernel,
        mesh=mesh,
        out_type=jax.ShapeDtypeStruct((NW * n_blk * _BLK, D), jnp.float32),
        scratch_types=[
            pltpu.VMEM((n_blk, _BLK), jnp.int32),
            pltpu.VMEM((2, _BLK, D), jnp.float32),
            pltpu.SemaphoreType.DMA,
            pltpu.SemaphoreType.DMA,
            pltpu.SemaphoreType.DMA,
            pltpu.SemaphoreType.DMA,
        ],
        compiler_params=pltpu.CompilerParams(use_tc_tiling_on_sc=False),
    )
    def gather(tab_hbm, idx_hbm, out_hbm, idx_v, rows_v, g0, g1, o0, o1):
        gsem = (g0, g1)
        osem = (o0, o1)
        wid = lax.axis_index("s") * NC + lax.axis_index("c")
        pltpu.sync_copy(idx_hbm.at[wid], idx_v)
        base_pair = wid * (n_blk * _BLK)
        pltpu.async_copy(tab_hbm.at[idx_v.at[0]], rows_v.at[0], g0)

        def step(j, s):
            pltpu.make_async_copy(
                tab_hbm.at[idx_v.at[j]], rows_v.at[s], gsem[s]
            ).wait()

            @pl.when(j >= 1)
            def _():
                pltpu.make_async_copy(
                    rows_v.at[1 - s],
                    out_hbm.at[pl.ds(base_pair + (j - 1) * _BLK, _BLK)],
                    osem[1 - s],
                ).wait()

            @pl.when(j < n_blk - 1)
            def _():
                pltpu.async_copy(
                    tab_hbm.at[idx_v.at[j + 1]], rows_v.at[1 - s], gsem[1 - s]
                )

            pltpu.async_copy(
                rows_v.at[s],
                out_hbm.at[pl.ds(base_pair + j * _BLK, _BLK)],
                osem[s],
            )

        def loop(i, carry):
            step(2 * i, 0)
            step(2 * i + 1, 1)
            return carry

        lax.fori_loop(0, n_blk // 2, loop, 0)
        pltpu.make_async_copy(
            rows_v.at[1],
            out_hbm.at[pl.ds(base_pair + (n_blk - 1) * _BLK, _BLK)],
            o1,
        ).wait()

    return gather


_BB = 256


def _trans_body(g_ref, eye_ref, o_ref):
    nlp = g_ref.shape[0] // _BB
    g3 = g_ref.reshape(_BB, nlp, g_ref.shape[1])
    eye = eye_ref[...]
    for lp in range(nlp):
        y = g3[:, lp, :]
        yt = lax.dot_general(
            y, eye, (((0,), (0,)), ((), ())),
            preferred_element_type=jnp.float32,
        )
        o_ref[2 * lp, :, :] = yt[:64, :]
        o_ref[2 * lp + 1, :, :] = yt[64:, :]


def _to_entry_layout(g2, B, L, D):
    eye = jnp.eye(_BB, dtype=jnp.float32)
    out3 = pl.pallas_call(
        _trans_body,
        grid=(B // _BB,),
        in_specs=[
            pl.BlockSpec((_BB * L // 2, 2 * D), lambda i: (i, 0)),
            pl.BlockSpec((_BB, _BB), lambda i: (0, 0)),
        ],
        out_specs=pl.BlockSpec((L, D, _BB), lambda i: (0, 0, i)),
        out_shape=jax.ShapeDtypeStruct((L, D, B), jnp.float32),
    )(g2, eye)
    return out3.transpose(2, 0, 1)


def kernel(color_indices, table, W, b):
    B, L = color_indices.shape
    V, D = table.shape
    info = plsc.get_sparse_core_info()
    NC, NS = info.num_cores, info.num_subcores
    NW = NC * NS
    total = B * L
    assert total % (NW * _BLK) == 0
    n_blk = total // (NW * _BLK)

    proj = _project_table(table, W, b)
    idx = color_indices.astype(jnp.int32).reshape(NW, n_blk, _BLK)
    g = _make_gather(V, D, NW, NC, n_blk)(proj, idx)
    g2 = g.reshape(B * L // 2, 2 * D)
    return _to_entry_layout(g2, B, L, D)


# speedup vs baseline: 1.2125x; 1.0998x over previous
import functools

import jax
import jax.numpy as jnp
from jax import lax
from jax.experimental import pallas as pl
from jax.experimental.pallas import tpu as pltpu
from jax.experimental.pallas import tpu_sc as plsc


def _proj_body(t_ref, wt_ref, b_ref, o_ref):
    o_ref[...] = (
        jnp.dot(t_ref[...], wt_ref[...], preferred_element_type=jnp.float32)
        + b_ref[...]
    )


def _project_table(table, W, b):
    V, D = table.shape
    t2 = table.reshape(V // 2, 2 * D)
    Wt = W.T
    W2 = jnp.zeros((2 * D, 2 * D), jnp.float32)
    W2 = W2.at[:D, :D].set(Wt).at[D:, D:].set(Wt)
    b2 = jnp.concatenate([b, b]).reshape(1, 2 * D)
    BLK = 2000
    assert (V // 2) % BLK == 0
    proj2 = pl.pallas_call(
        _proj_body,
        grid=(V // 2 // BLK,),
        in_specs=[
            pl.BlockSpec((BLK, 2 * D), lambda i: (i, 0)),
            pl.BlockSpec((2 * D, 2 * D), lambda i: (0, 0)),
            pl.BlockSpec((1, 2 * D), lambda i: (0, 0)),
        ],
        out_specs=pl.BlockSpec((BLK, 2 * D), lambda i: (i, 0)),
        out_shape=jax.ShapeDtypeStruct((V // 2, 2 * D), jnp.float32),
    )(t2, W2, b2)
    return proj2.reshape(V, D)


_BLK = 128


def _make_gather(V, D, NW, NC, n_blk):
    mesh = plsc.VectorSubcoreMesh(core_axis_name="c", subcore_axis_name="s")

    @functools.partial(
        pl.kernel,
        mesh=mesh,
        out_type=jax.ShapeDtypeStruct((NW * n_blk * _BLK, D), jnp.float32),
        scratch_types=[
            pltpu.VMEM((n_blk, _BLK), jnp.int32),
            pltpu.VMEM((3, _BLK, D), jnp.float32),
            pltpu.SemaphoreType.DMA,
            pltpu.SemaphoreType.DMA,
            pltpu.SemaphoreType.DMA,
            pltpu.SemaphoreType.DMA,
            pltpu.SemaphoreType.DMA,
            pltpu.SemaphoreType.DMA,
        ],
        compiler_params=pltpu.CompilerParams(use_tc_tiling_on_sc=False),
    )
    def gather(tab_hbm, idx_hbm, out_hbm, idx_v, rows_v,
               g0, g1, g2s, o0, o1, o2s):
        gsem = (g0, g1, g2s)
        osem = (o0, o1, o2s)
        wid = lax.axis_index("s") * NC + lax.axis_index("c")
        pltpu.sync_copy(idx_hbm.at[wid], idx_v)
        base_pair = wid * (n_blk * _BLK)
        pltpu.async_copy(tab_hbm.at[idx_v.at[0]], rows_v.at[0], g0)
        pltpu.async_copy(tab_hbm.at[idx_v.at[1]], rows_v.at[1], g1)

        def step(j, s):
            pltpu.make_async_copy(
                tab_hbm.at[idx_v.at[j]], rows_v.at[s], gsem[s]
            ).wait()

            sn = (s + 2) % 3
            @pl.when(j >= 1)
            def _():
                pltpu.make_async_copy(
                    rows_v.at[sn],
                    out_hbm.at[pl.ds(base_pair + (j - 1) * _BLK, _BLK)],
                    osem[sn],
                ).wait()

            @pl.when(j < n_blk - 2)
            def _():
                pltpu.async_copy(
                    tab_hbm.at[idx_v.at[j + 2]], rows_v.at[sn], gsem[sn]
                )

            pltpu.async_copy(
                rows_v.at[s],
                out_hbm.at[pl.ds(base_pair + j * _BLK, _BLK)],
                osem[s],
            )

        def loop(i, carry):
            step(3 * i, 0)
            step(3 * i + 1, 1)
            step(3 * i + 2, 2)
            return carry

        nfull = (n_blk - 2) // 3
        lax.fori_loop(0, nfull, loop, 0)
        for j in range(3 * nfull, n_blk):
            step(j, j % 3)
        pltpu.make_async_copy(
            rows_v.at[(n_blk - 1) % 3],
            out_hbm.at[pl.ds(base_pair + (n_blk - 1) * _BLK, _BLK)],
            osem[(n_blk - 1) % 3],
        ).wait()

    return gather


_BB = 256


def _trans_body(g_ref, eye_ref, o_ref):
    nlp = g_ref.shape[0] // _BB
    g3 = g_ref.reshape(_BB, nlp, g_ref.shape[1])
    eye = eye_ref[...]
    for lp in range(nlp):
        y = g3[:, lp, :]
        yt = lax.dot_general(
            y, eye, (((0,), (0,)), ((), ())),
            preferred_element_type=jnp.float32,
        )
        o_ref[2 * lp, :, :] = yt[:64, :]
        o_ref[2 * lp + 1, :, :] = yt[64:, :]


def _to_entry_layout(g2, B, L, D):
    eye = jnp.eye(_BB, dtype=jnp.float32)
    out3 = pl.pallas_call(
        _trans_body,
        grid=(B // _BB,),
        in_specs=[
            pl.BlockSpec((_BB * L // 2, 2 * D), lambda i: (i, 0)),
            pl.BlockSpec((_BB, _BB), lambda i: (0, 0)),
        ],
        out_specs=pl.BlockSpec((L, D, _BB), lambda i: (0, 0, i)),
        out_shape=jax.ShapeDtypeStruct((L, D, B), jnp.float32),
    )(g2, eye)
    return out3.transpose(2, 0, 1)


def kernel(color_indices, table, W, b):
    B, L = color_indices.shape
    V, D = table.shape
    info = plsc.get_sparse_core_info()
    NC, NS = info.num_cores, info.num_subcores
    NW = NC * NS
    total = B * L
    assert total % (NW * _BLK) == 0
    n_blk = total // (NW * _BLK)

    proj = _project_table(table, W, b)
    idx = color_indices.astype(jnp.int32).reshape(NW, n_blk, _BLK)
    g = _make_gather(V, D, NW, NC, n_blk)(proj, idx)
    g2 = g.reshape(B * L // 2, 2 * D)
    return _to_entry_layout(g2, B, L, D)


# speedup vs baseline: 1.2165x; 1.0033x over previous
import functools

import jax
import jax.numpy as jnp
from jax import lax
from jax.experimental import pallas as pl
from jax.experimental.pallas import tpu as pltpu
from jax.experimental.pallas import tpu_sc as plsc


def _proj_body(t_ref, wt_ref, b_ref, o_ref):
    o_ref[...] = (
        jnp.dot(t_ref[...], wt_ref[...], preferred_element_type=jnp.float32)
        + b_ref[...]
    )


def _project_table(table, W, b):
    V, D = table.shape
    t2 = table.reshape(V // 2, 2 * D)
    Wt = W.T
    W2 = jnp.zeros((2 * D, 2 * D), jnp.float32)
    W2 = W2.at[:D, :D].set(Wt).at[D:, D:].set(Wt)
    b2 = jnp.concatenate([b, b]).reshape(1, 2 * D)
    BLK = 2000
    assert (V // 2) % BLK == 0
    proj2 = pl.pallas_call(
        _proj_body,
        grid=(V // 2 // BLK,),
        in_specs=[
            pl.BlockSpec((BLK, 2 * D), lambda i: (i, 0)),
            pl.BlockSpec((2 * D, 2 * D), lambda i: (0, 0)),
            pl.BlockSpec((1, 2 * D), lambda i: (0, 0)),
        ],
        out_specs=pl.BlockSpec((BLK, 2 * D), lambda i: (i, 0)),
        out_shape=jax.ShapeDtypeStruct((V // 2, 2 * D), jnp.float32),
    )(t2, W2, b2)
    return proj2.reshape(V, D)


_BLK = 128


def _make_gather(V, D, NW, NC, n_blk):
    mesh = plsc.VectorSubcoreMesh(core_axis_name="c", subcore_axis_name="s")

    @functools.partial(
        pl.kernel,
        mesh=mesh,
        out_type=jax.ShapeDtypeStruct((NW * n_blk * _BLK, D), jnp.float32),
        scratch_types=[
            pltpu.VMEM((n_blk, _BLK), jnp.int32),
            pltpu.VMEM((4, _BLK, D), jnp.float32),
            pltpu.SemaphoreType.DMA,
            pltpu.SemaphoreType.DMA,
            pltpu.SemaphoreType.DMA,
            pltpu.SemaphoreType.DMA,
            pltpu.SemaphoreType.DMA,
            pltpu.SemaphoreType.DMA,
            pltpu.SemaphoreType.DMA,
            pltpu.SemaphoreType.DMA,
        ],
        compiler_params=pltpu.CompilerParams(use_tc_tiling_on_sc=False),
    )
    def gather(tab_hbm, idx_hbm, out_hbm, idx_v, rows_v,
               g0, g1, g2s, g3s, o0, o1, o2s, o3s):
        gsem = (g0, g1, g2s, g3s)
        osem = (o0, o1, o2s, o3s)
        wid = lax.axis_index("s") * NC + lax.axis_index("c")
        pltpu.sync_copy(idx_hbm.at[wid], idx_v)
        base_pair = wid * (n_blk * _BLK)
        pltpu.async_copy(tab_hbm.at[idx_v.at[0]], rows_v.at[0], g0)
        pltpu.async_copy(tab_hbm.at[idx_v.at[1]], rows_v.at[1], g1)

        def step(j, s):
            pltpu.make_async_copy(
                tab_hbm.at[idx_v.at[j]], rows_v.at[s], gsem[s]
            ).wait()

            sn = (s + 2) % 4
            @pl.when(j >= 2)
            def _():
                pltpu.make_async_copy(
                    rows_v.at[sn],
                    out_hbm.at[pl.ds(base_pair + (j - 2) * _BLK, _BLK)],
                    osem[sn],
                ).wait()

            @pl.when(j < n_blk - 2)
            def _():
                pltpu.async_copy(
                    tab_hbm.at[idx_v.at[j + 2]], rows_v.at[sn], gsem[sn]
                )

            pltpu.async_copy(
                rows_v.at[s],
                out_hbm.at[pl.ds(base_pair + j * _BLK, _BLK)],
                osem[s],
            )

        def loop(i, carry):
            step(4 * i, 0)
            step(4 * i + 1, 1)
            step(4 * i + 2, 2)
            step(4 * i + 3, 3)
            return carry

        nfull = (n_blk - 2) // 4
        lax.fori_loop(0, nfull, loop, 0)
        for j in range(4 * nfull, n_blk):
            step(j, j % 4)
        for j in (n_blk - 2, n_blk - 1):
            pltpu.make_async_copy(
                rows_v.at[j % 4],
                out_hbm.at[pl.ds(base_pair + j * _BLK, _BLK)],
                osem[j % 4],
            ).wait()

    return gather


_BB = 256


def _trans_body(g_ref, eye_ref, o_ref):
    nlp = g_ref.shape[0] // _BB
    g3 = g_ref.reshape(_BB, nlp, g_ref.shape[1])
    eye = eye_ref[...]
    for lp in range(nlp):
        y = g3[:, lp, :]
        yt = lax.dot_general(
            y, eye, (((0,), (0,)), ((), ())),
            preferred_element_type=jnp.float32,
        )
        o_ref[2 * lp, :, :] = yt[:64, :]
        o_ref[2 * lp + 1, :, :] = yt[64:, :]


def _to_entry_layout(g2, B, L, D):
    eye = jnp.eye(_BB, dtype=jnp.float32)
    out3 = pl.pallas_call(
        _trans_body,
        grid=(B // _BB,),
        in_specs=[
            pl.BlockSpec((_BB * L // 2, 2 * D), lambda i: (i, 0)),
            pl.BlockSpec((_BB, _BB), lambda i: (0, 0)),
        ],
        out_specs=pl.BlockSpec((L, D, _BB), lambda i: (0, 0, i)),
        out_shape=jax.ShapeDtypeStruct((L, D, B), jnp.float32),
    )(g2, eye)
    return out3.transpose(2, 0, 1)


def kernel(color_indices, table, W, b):
    B, L = color_indices.shape
    V, D = table.shape
    info = plsc.get_sparse_core_info()
    NC, NS = info.num_cores, info.num_subcores
    NW = NC * NS
    total = B * L
    assert total % (NW * _BLK) == 0
    n_blk = total // (NW * _BLK)

    proj = _project_table(table, W, b)
    idx = color_indices.astype(jnp.int32).reshape(NW, n_blk, _BLK)
    g = _make_gather(V, D, NW, NC, n_blk)(proj, idx)
    g2 = g.reshape(B * L // 2, 2 * D)
    return _to_entry_layout(g2, B, L, D)
